# baseline (speedup 1.0000x reference)
import functools

import jax
import jax.numpy as jnp
from jax import lax
from jax.experimental import pallas as pl
from jax.experimental.pallas import tpu as pltpu
from jax.experimental.pallas import tpu_sc as plsc

NC, NS = 2, 16
NW = NC * NS
S = 2
EBLK = 8000
NBLK = 1000

def _sc_mesh():
    return plsc.VectorSubcoreMesh(core_axis_name="c", subcore_axis_name="s",
                                  num_cores=NC)


def _prep_body(nf_ref, w0a_ref, w0b_ref, b0_ref, g_ref, wr_ref):
    g_ref[...] = (
        jnp.dot(nf_ref[...], w0a_ref[...], preferred_element_type=jnp.float32)
        + b0_ref[...]
    )
    wr_ref[...] = jnp.sum(w0b_ref[...], axis=0, keepdims=True)


def _edge_body(gs_ref, ef_ref, c_ref, w0c_ref, w1_ref, w2_ref, vec_ref,
               ne_ref, eo_ref):
    ef = ef_ref[...]
    b1 = vec_ref[0:1, :]
    b2 = vec_ref[1:2, :]
    g = vec_ref[2:3, :]
    beta = vec_ref[3:4, :]
    wr = vec_ref[4:5, :]
    c2 = jnp.reshape(c_ref[...], (1, ef.shape[0]))
    couter = lax.dot_general(c2, wr, (((0,), (0,)), ((), ())),
                             preferred_element_type=jnp.float32)
    y = gs_ref[...] + couter
    y = y + jnp.dot(ef, w0c_ref[...], preferred_element_type=jnp.float32)
    y = jnp.maximum(y, 0.0)
    y = jnp.dot(y, w1_ref[...], preferred_element_type=jnp.float32) + b1
    y = jnp.maximum(y, 0.0)
    y = jnp.dot(y, w2_ref[...], preferred_element_type=jnp.float32) + b2
    mu = jnp.mean(y, axis=-1, keepdims=True)
    yc = y - mu
    var = jnp.mean(yc * yc, axis=-1, keepdims=True)
    ne = yc * lax.rsqrt(var + 1e-5) * g + beta
    ne_ref[...] = ne
    eo_ref[...] = ne + ef


def _node_body(nf_ref, a0_ref, a1_ref, a2_ref, a3_ref, wn0a_ref, wn0b_ref,
               wn1_ref, wn2_ref, vec_ref, out_ref):
    nf = nf_ref[...]
    acc = (a0_ref[...] + a1_ref[...]) + (a2_ref[...] + a3_ref[...])
    b0 = vec_ref[0:1, :]
    b1 = vec_ref[1:2, :]
    b2 = vec_ref[2:3, :]
    g = vec_ref[3:4, :]
    beta = vec_ref[4:5, :]
    y = (jnp.dot(nf, wn0a_ref[...], preferred_element_type=jnp.float32)
         + jnp.dot(acc, wn0b_ref[...], preferred_element_type=jnp.float32)
         + b0)
    y = jnp.maximum(y, 0.0)
    y = jnp.dot(y, wn1_ref[...], preferred_element_type=jnp.float32) + b1
    y = jnp.maximum(y, 0.0)
    y = jnp.dot(y, wn2_ref[...], preferred_element_type=jnp.float32) + b2
    mu = jnp.mean(y, axis=-1, keepdims=True)
    yc = y - mu
    var = jnp.mean(yc * yc, axis=-1, keepdims=True)
    out_ref[...] = yc * lax.rsqrt(var + 1e-5) * g + beta + nf


def _make_sc_gather(n, e, h, ept, nchunk, CHUNK):
    @functools.partial(
        pl.kernel,
        mesh=_sc_mesh(),
        compiler_params=pltpu.CompilerParams(needs_layout_passes=False),
        out_type=(jax.ShapeDtypeStruct((e, h), jnp.float32),
                  jax.ShapeDtypeStruct((e,), jnp.float32)),
        scratch_types=[
            pltpu.VMEM((nchunk, CHUNK), jnp.int32),
            pltpu.VMEM((ept,), jnp.int32),
            pltpu.VMEM((n,), jnp.int32),
            pltpu.VMEM((ept,), jnp.float32),
            pltpu.VMEM((CHUNK, h), jnp.float32),
            pltpu.VMEM((CHUNK, h), jnp.float32),
            pltpu.SemaphoreType.DMA,
            pltpu.SemaphoreType.DMA,
        ],
    )
    def sc_gather(g_hbm, sidx_hbm, ridx_hbm, rtab_hbm, gs_out, c_out,
                  sidx_v, ridx_v, rtab_v, c_v, rows0_v, rows1_v, sem0, sem1):
        wid = lax.axis_index("s") * NC + lax.axis_index("c")
        ebase = wid * ept
        pltpu.sync_copy(sidx_hbm.at[wid], sidx_v)
        pltpu.async_copy(g_hbm.at[sidx_v.at[0]], rows0_v, sem0)
        pltpu.sync_copy(ridx_hbm.at[pl.ds(ebase, ept)], ridx_v)
        pltpu.sync_copy(rtab_hbm, rtab_v)

        def c_step(i, carry):
            idx = ridx_v[pl.ds(i * 16, 16)]
            vals = plsc.load_gather(rtab_v, [idx])
            c_v[pl.ds(i * 16, 16)] = vals.astype(jnp.float32)
            return carry

        lax.fori_loop(0, ept // 16, c_step, 0)
        pltpu.sync_copy(c_v, c_out.at[pl.ds(ebase, ept)])

        def out_at(j):
            return gs_out.at[pl.ds(ebase + j * CHUNK, CHUNK)]

        def g_pair(jj, carry):
            j0 = jj * 2
            pltpu.make_async_copy(g_hbm.at[sidx_v.at[j0]], rows0_v,
                                  sem0).wait()
            pltpu.async_copy(g_hbm.at[sidx_v.at[j0 + 1]], rows1_v, sem1)
            pltpu.sync_copy(rows0_v, out_at(j0))
            pltpu.make_async_copy(g_hbm.at[sidx_v.at[j0 + 1]], rows1_v,
                                  sem1).wait()

            @pl.when(j0 + 2 < nchunk)
            def _():
                pltpu.async_copy(g_hbm.at[sidx_v.at[j0 + 2]], rows0_v, sem0)

            pltpu.sync_copy(rows1_v, out_at(j0 + 1))
            return carry

        lax.fori_loop(0, nchunk // 2, g_pair, 0)
        if nchunk % 2 == 1:
            j_last = nchunk - 1
            pltpu.make_async_copy(g_hbm.at[sidx_v.at[j_last]], rows0_v,
                                  sem0).wait()
            pltpu.sync_copy(rows0_v, out_at(j_last))

    return sc_gather


def _make_sc_scatter(n_pad, e, h, ept, nchunk, npt, CHUNK):
    @functools.partial(
        pl.kernel,
        mesh=_sc_mesh(),
        compiler_params=pltpu.CompilerParams(needs_layout_passes=False),
        out_type=jax.ShapeDtypeStruct((NC, n_pad, h), jnp.float32),
        scratch_types=[
            pltpu.VMEM((nchunk, CHUNK), jnp.int32),
            pltpu.VMEM((CHUNK, h), jnp.float32),
            pltpu.VMEM((CHUNK, h), jnp.float32),
            pltpu.VMEM_SHARED((n_pad, h), jnp.float32),
            pltpu.SemaphoreType.DMA,
            pltpu.SemaphoreType.DMA,
        ],
    )
    def sc_scatter(ne_hbm, ridx_hbm, zeros_hbm, out_hbm, idx_v, buf0_v, buf1_v,
                   acc_sh, sem0, sem1):
        cid = lax.axis_index("c")
        sid = lax.axis_index("s")
        wid = sid * NC + cid
        ebase = wid * ept
        pltpu.sync_copy(ridx_hbm.at[wid], idx_v)

        def ne_at(j):
            return ne_hbm.at[pl.ds(ebase + j * CHUNK, CHUNK)]

        pltpu.async_copy(ne_at(0), buf0_v, sem0)
        pltpu.sync_copy(zeros_hbm.at[pl.ds(sid * npt, npt)],
                        acc_sh.at[pl.ds(sid * npt, npt)])
        plsc.subcore_barrier()

        def s_pair(jj, carry):
            j0 = jj * 2
            pltpu.make_async_copy(ne_at(j0), buf0_v, sem0).wait()
            pltpu.async_copy(ne_at(j0 + 1), buf1_v, sem1)
            pltpu.sync_copy(buf0_v, acc_sh.at[idx_v.at[j0]], add=True)
            pltpu.make_async_copy(ne_at(j0 + 1), buf1_v, sem1).wait()

            @pl.when(j0 + 2 < nchunk)
            def _():
                pltpu.async_copy(ne_at(j0 + 2), buf0_v, sem0)

            pltpu.sync_copy(buf1_v, acc_sh.at[idx_v.at[j0 + 1]], add=True)
            return carry

        lax.fori_loop(0, nchunk // 2, s_pair, 0)
        if nchunk % 2 == 1:
            j_last = nchunk - 1
            pltpu.make_async_copy(ne_at(j_last), buf0_v, sem0).wait()
            pltpu.sync_copy(buf0_v, acc_sh.at[idx_v.at[j_last]], add=True)
        plsc.subcore_barrier()
        pltpu.sync_copy(acc_sh.at[pl.ds(sid * npt, npt)],
                        out_hbm.at[cid, pl.ds(sid * npt, npt)])

    return sc_scatter


def kernel(senders, receivers, node_features, edge_features, params):
    b, n, h = node_features.shape
    e = senders.shape[1]
    es = e // S
    ept = es // NW
    chunk = 80 if ept % 80 == 0 else 40
    nchunk = ept // chunk
    npt = n // NS

    s = senders.reshape(e).astype(jnp.int32)
    r = receivers.reshape(e).astype(jnp.int32)
    nf = node_features.reshape(n, h)
    ef = edge_features.reshape(e, h)
    p = params

    w0 = p["edge_W0"]
    w0a, w0b, w0c = w0[:h], w0[h:2 * h], w0[2 * h:]

    prep = pl.pallas_call(
        _prep_body,
        grid=(n // NBLK,),
        in_specs=[
            pl.BlockSpec((NBLK, h), lambda i: (i, 0)),
            pl.BlockSpec((h, h), lambda i: (0, 0)),
            pl.BlockSpec((h, h), lambda i: (0, 0)),
            pl.BlockSpec((1, h), lambda i: (0, 0)),
        ],
        out_specs=[
            pl.BlockSpec((NBLK, h), lambda i: (i, 0)),
            pl.BlockSpec((1, h), lambda i: (0, 0)),
        ],
        out_shape=[
            jax.ShapeDtypeStruct((n, h), jnp.float32),
            jax.ShapeDtypeStruct((1, h), jnp.float32),
        ],
    )
    g_tab, wr = prep(nf, w0a, w0b, p["edge_b0"].reshape(1, h))

    rtab = r[:n]
    evecs = jnp.concatenate([
        p["edge_b1"].reshape(1, h), p["edge_b2"].reshape(1, h),
        p["edge_g"].reshape(1, h), p["edge_beta"].reshape(1, h),
        wr, jnp.zeros((3, h), jnp.float32),
    ], axis=0)

    sc_gather = _make_sc_gather(n, es, h, ept, nchunk, chunk)
    npt_pad = -(-npt // 8) * 8
    n_pad = NS * npt_pad
    zeros = jnp.zeros((n_pad, h), jnp.float32)
    sc_scatter = _make_sc_scatter(n_pad, es, h, ept, nchunk, npt_pad, chunk)

    nb_s = es // EBLK

    def make_edge_mlp(k):
        return pl.pallas_call(
            _edge_body,
            grid=(nb_s,),
            in_specs=[
                pl.BlockSpec((EBLK, h), lambda i: (i, 0)),
                pl.BlockSpec((EBLK, h), lambda i, k=k: (k * nb_s + i, 0)),
                pl.BlockSpec((1, 1, EBLK), lambda i: (i, 0, 0)),
                pl.BlockSpec((h, h), lambda i: (0, 0)),
                pl.BlockSpec((h, h), lambda i: (0, 0)),
                pl.BlockSpec((h, h), lambda i: (0, 0)),
                pl.BlockSpec((8, h), lambda i: (0, 0)),
            ],
            out_specs=[
                pl.BlockSpec((EBLK, h), lambda i: (i, 0)),
                pl.BlockSpec((EBLK, h), lambda i: (i, 0)),
            ],
            out_shape=[
                jax.ShapeDtypeStruct((es, h), jnp.float32),
                jax.ShapeDtypeStruct((es, h), jnp.float32),
            ],
        )

    gs_c = []
    for k in range(S):
        sk = lax.slice(s, (k * es,), ((k + 1) * es,)).reshape(NW, nchunk,
                                                              chunk)
        rk = lax.slice(r, (k * es,), ((k + 1) * es,))
        gs_c.append(sc_gather(g_tab, sk, rk, rtab))

    parts, eos = [], []
    for k in range(S):
        gs_k, c_k = gs_c[k]
        c3_k = c_k.reshape(nb_s, 1, EBLK)
        ne_k, eo_k = make_edge_mlp(k)(gs_k, ef, c3_k, w0c, p["edge_W1"],
                                      p["edge_W2"], evecs)
        eos.append(eo_k)
        r3_k = lax.slice(r, (k * es,), ((k + 1) * es,)).reshape(NW, nchunk,
                                                                chunk)
        parts.append(sc_scatter(ne_k, r3_k, zeros))

    eo = jnp.concatenate(eos, axis=0)
    accs = [lax.slice(part[i], (0, 0), (n, h))
            for part in parts for i in range(NC)]

    wn0 = p["node_W0"]
    wn0a, wn0b = wn0[:h], wn0[h:]
    nvecs = jnp.concatenate([
        p["node_b0"].reshape(1, h), p["node_b1"].reshape(1, h),
        p["node_b2"].reshape(1, h), p["node_g"].reshape(1, h),
        p["node_beta"].reshape(1, h), jnp.zeros((3, h), jnp.float32),
    ], axis=0)

    node_mlp = pl.pallas_call(
        _node_body,
        grid=(n // NBLK,),
        in_specs=[
            pl.BlockSpec((NBLK, h), lambda i: (i, 0)),
            pl.BlockSpec((NBLK, h), lambda i: (i, 0)),
            pl.BlockSpec((NBLK, h), lambda i: (i, 0)),
            pl.BlockSpec((NBLK, h), lambda i: (i, 0)),
            pl.BlockSpec((NBLK, h), lambda i: (i, 0)),
            pl.BlockSpec((h, h), lambda i: (0, 0)),
            pl.BlockSpec((h, h), lambda i: (0, 0)),
            pl.BlockSpec((h, h), lambda i: (0, 0)),
            pl.BlockSpec((h, h), lambda i: (0, 0)),
            pl.BlockSpec((8, h), lambda i: (0, 0)),
        ],
        out_specs=pl.BlockSpec((NBLK, h), lambda i: (i, 0)),
        out_shape=jax.ShapeDtypeStruct((n, h), jnp.float32),
    )
    nn = node_mlp(nf, accs[0], accs[1], accs[2], accs[3], wn0a, wn0b,
                  p["node_W1"], p["node_W2"], nvecs)

    return nn.reshape(b, n, h), eo.reshape(b, e, h)


# speedup vs baseline: 1.2746x; 1.2746x over previous
import functools

import jax
import jax.numpy as jnp
from jax import lax
from jax.experimental import pallas as pl
from jax.experimental.pallas import tpu as pltpu
from jax.experimental.pallas import tpu_sc as plsc

NC, NS = 2, 16
NW = NC * NS
S = 1
EBLK = 8000
NBLK = 1000

def _sc_mesh():
    return plsc.VectorSubcoreMesh(core_axis_name="c", subcore_axis_name="s",
                                  num_cores=NC)


def _prep_body(nf_ref, w0a_ref, w0b_ref, b0_ref, g_ref, wr_ref):
    g_ref[...] = (
        jnp.dot(nf_ref[...], w0a_ref[...], preferred_element_type=jnp.float32)
        + b0_ref[...]
    )
    wr_ref[...] = jnp.sum(w0b_ref[...], axis=0, keepdims=True)


def _edge_body(gs_ref, ef_ref, c_ref, w0c_ref, w1_ref, w2_ref, vec_ref,
               ne_ref, eo_ref):
    ef = ef_ref[...]
    b1 = vec_ref[0:1, :]
    b2 = vec_ref[1:2, :]
    g = vec_ref[2:3, :]
    beta = vec_ref[3:4, :]
    wr = vec_ref[4:5, :]
    c2 = jnp.reshape(c_ref[...], (1, ef.shape[0]))
    couter = lax.dot_general(c2, wr, (((0,), (0,)), ((), ())),
                             preferred_element_type=jnp.float32)
    y = gs_ref[...] + couter
    y = y + jnp.dot(ef, w0c_ref[...], preferred_element_type=jnp.float32)
    y = jnp.maximum(y, 0.0)
    y = jnp.dot(y, w1_ref[...], preferred_element_type=jnp.float32) + b1
    y = jnp.maximum(y, 0.0)
    y = jnp.dot(y, w2_ref[...], preferred_element_type=jnp.float32) + b2
    mu = jnp.mean(y, axis=-1, keepdims=True)
    yc = y - mu
    var = jnp.mean(yc * yc, axis=-1, keepdims=True)
    ne = yc * lax.rsqrt(var + 1e-5) * g + beta
    ne_ref[...] = ne
    eo_ref[...] = ne + ef


def _node_body(*refs):
    nf_ref = refs[0]
    acc_refs = refs[1:-6]
    wn0a_ref, wn0b_ref, wn1_ref, wn2_ref, vec_ref, out_ref = refs[-6:]
    nf = nf_ref[...]
    acc = acc_refs[0][...]
    for a_ref in acc_refs[1:]:
        acc = acc + a_ref[...]
    b0 = vec_ref[0:1, :]
    b1 = vec_ref[1:2, :]
    b2 = vec_ref[2:3, :]
    g = vec_ref[3:4, :]
    beta = vec_ref[4:5, :]
    y = (jnp.dot(nf, wn0a_ref[...], preferred_element_type=jnp.float32)
         + jnp.dot(acc, wn0b_ref[...], preferred_element_type=jnp.float32)
         + b0)
    y = jnp.maximum(y, 0.0)
    y = jnp.dot(y, wn1_ref[...], preferred_element_type=jnp.float32) + b1
    y = jnp.maximum(y, 0.0)
    y = jnp.dot(y, wn2_ref[...], preferred_element_type=jnp.float32) + b2
    mu = jnp.mean(y, axis=-1, keepdims=True)
    yc = y - mu
    var = jnp.mean(yc * yc, axis=-1, keepdims=True)
    out_ref[...] = yc * lax.rsqrt(var + 1e-5) * g + beta + nf


def _make_sc_gather(n, e, h, ept, nchunk, CHUNK):
    @functools.partial(
        pl.kernel,
        mesh=_sc_mesh(),
        compiler_params=pltpu.CompilerParams(needs_layout_passes=False),
        out_type=(jax.ShapeDtypeStruct((e, h), jnp.float32),
                  jax.ShapeDtypeStruct((e,), jnp.float32)),
        scratch_types=[
            pltpu.VMEM((nchunk, CHUNK), jnp.int32),
            pltpu.VMEM((ept,), jnp.int32),
            pltpu.VMEM((n,), jnp.int32),
            pltpu.VMEM((ept,), jnp.float32),
            pltpu.VMEM((CHUNK, h), jnp.float32),
            pltpu.VMEM((CHUNK, h), jnp.float32),
            pltpu.SemaphoreType.DMA,
            pltpu.SemaphoreType.DMA,
        ],
    )
    def sc_gather(g_hbm, sidx_hbm, ridx_hbm, rtab_hbm, gs_out, c_out,
                  sidx_v, ridx_v, rtab_v, c_v, rows0_v, rows1_v, sem0, sem1):
        wid = lax.axis_index("s") * NC + lax.axis_index("c")
        ebase = wid * ept
        pltpu.sync_copy(sidx_hbm.at[wid], sidx_v)
        pltpu.async_copy(g_hbm.at[sidx_v.at[0]], rows0_v, sem0)
        pltpu.sync_copy(ridx_hbm.at[pl.ds(ebase, ept)], ridx_v)
        pltpu.sync_copy(rtab_hbm, rtab_v)

        def c_step(i, carry):
            idx = ridx_v[pl.ds(i * 16, 16)]
            vals = plsc.load_gather(rtab_v, [idx])
            c_v[pl.ds(i * 16, 16)] = vals.astype(jnp.float32)
            return carry

        lax.fori_loop(0, ept // 16, c_step, 0)
        pltpu.sync_copy(c_v, c_out.at[pl.ds(ebase, ept)])

        def out_at(j):
            return gs_out.at[pl.ds(ebase + j * CHUNK, CHUNK)]

        def g_pair(jj, carry):
            j0 = jj * 2
            pltpu.make_async_copy(g_hbm.at[sidx_v.at[j0]], rows0_v,
                                  sem0).wait()
            pltpu.async_copy(g_hbm.at[sidx_v.at[j0 + 1]], rows1_v, sem1)
            pltpu.sync_copy(rows0_v, out_at(j0))
            pltpu.make_async_copy(g_hbm.at[sidx_v.at[j0 + 1]], rows1_v,
                                  sem1).wait()

            @pl.when(j0 + 2 < nchunk)
            def _():
                pltpu.async_copy(g_hbm.at[sidx_v.at[j0 + 2]], rows0_v, sem0)

            pltpu.sync_copy(rows1_v, out_at(j0 + 1))
            return carry

        lax.fori_loop(0, nchunk // 2, g_pair, 0)
        if nchunk % 2 == 1:
            j_last = nchunk - 1
            pltpu.make_async_copy(g_hbm.at[sidx_v.at[j_last]], rows0_v,
                                  sem0).wait()
            pltpu.sync_copy(rows0_v, out_at(j_last))

    return sc_gather


def _make_sc_scatter(n_pad, e, h, ept, nchunk, npt, CHUNK):
    @functools.partial(
        pl.kernel,
        mesh=_sc_mesh(),
        compiler_params=pltpu.CompilerParams(needs_layout_passes=False),
        out_type=jax.ShapeDtypeStruct((NC, n_pad, h), jnp.float32),
        scratch_types=[
            pltpu.VMEM((nchunk, CHUNK), jnp.int32),
            pltpu.VMEM((CHUNK, h), jnp.float32),
            pltpu.VMEM((CHUNK, h), jnp.float32),
            pltpu.VMEM_SHARED((n_pad, h), jnp.float32),
            pltpu.SemaphoreType.DMA,
            pltpu.SemaphoreType.DMA,
        ],
    )
    def sc_scatter(ne_hbm, ridx_hbm, zeros_hbm, out_hbm, idx_v, buf0_v, buf1_v,
                   acc_sh, sem0, sem1):
        cid = lax.axis_index("c")
        sid = lax.axis_index("s")
        wid = sid * NC + cid
        ebase = wid * ept
        pltpu.sync_copy(ridx_hbm.at[wid], idx_v)

        def ne_at(j):
            return ne_hbm.at[pl.ds(ebase + j * CHUNK, CHUNK)]

        pltpu.async_copy(ne_at(0), buf0_v, sem0)
        pltpu.sync_copy(zeros_hbm.at[pl.ds(sid * npt, npt)],
                        acc_sh.at[pl.ds(sid * npt, npt)])
        plsc.subcore_barrier()

        def s_pair(jj, carry):
            j0 = jj * 2
            pltpu.make_async_copy(ne_at(j0), buf0_v, sem0).wait()
            pltpu.async_copy(ne_at(j0 + 1), buf1_v, sem1)
            pltpu.sync_copy(buf0_v, acc_sh.at[idx_v.at[j0]], add=True)
            pltpu.make_async_copy(ne_at(j0 + 1), buf1_v, sem1).wait()

            @pl.when(j0 + 2 < nchunk)
            def _():
                pltpu.async_copy(ne_at(j0 + 2), buf0_v, sem0)

            pltpu.sync_copy(buf1_v, acc_sh.at[idx_v.at[j0 + 1]], add=True)
            return carry

        lax.fori_loop(0, nchunk // 2, s_pair, 0)
        if nchunk % 2 == 1:
            j_last = nchunk - 1
            pltpu.make_async_copy(ne_at(j_last), buf0_v, sem0).wait()
            pltpu.sync_copy(buf0_v, acc_sh.at[idx_v.at[j_last]], add=True)
        plsc.subcore_barrier()
        pltpu.sync_copy(acc_sh.at[pl.ds(sid * npt, npt)],
                        out_hbm.at[cid, pl.ds(sid * npt, npt)])

    return sc_scatter


def kernel(senders, receivers, node_features, edge_features, params):
    b, n, h = node_features.shape
    e = senders.shape[1]
    es = e // S
    ept = es // NW
    chunk = 80 if ept % 80 == 0 else 40
    nchunk = ept // chunk
    npt = n // NS

    s = senders.reshape(e).astype(jnp.int32)
    r = receivers.reshape(e).astype(jnp.int32)
    nf = node_features.reshape(n, h)
    ef = edge_features.reshape(e, h)
    p = params

    w0 = p["edge_W0"]
    w0a, w0b, w0c = w0[:h], w0[h:2 * h], w0[2 * h:]

    prep = pl.pallas_call(
        _prep_body,
        grid=(n // NBLK,),
        in_specs=[
            pl.BlockSpec((NBLK, h), lambda i: (i, 0)),
            pl.BlockSpec((h, h), lambda i: (0, 0)),
            pl.BlockSpec((h, h), lambda i: (0, 0)),
            pl.BlockSpec((1, h), lambda i: (0, 0)),
        ],
        out_specs=[
            pl.BlockSpec((NBLK, h), lambda i: (i, 0)),
            pl.BlockSpec((1, h), lambda i: (0, 0)),
        ],
        out_shape=[
            jax.ShapeDtypeStruct((n, h), jnp.float32),
            jax.ShapeDtypeStruct((1, h), jnp.float32),
        ],
    )
    g_tab, wr = prep(nf, w0a, w0b, p["edge_b0"].reshape(1, h))

    rtab = r[:n]
    evecs = jnp.concatenate([
        p["edge_b1"].reshape(1, h), p["edge_b2"].reshape(1, h),
        p["edge_g"].reshape(1, h), p["edge_beta"].reshape(1, h),
        wr, jnp.zeros((3, h), jnp.float32),
    ], axis=0)

    sc_gather = _make_sc_gather(n, es, h, ept, nchunk, chunk)
    npt_pad = -(-npt // 8) * 8
    n_pad = NS * npt_pad
    zeros = jnp.zeros((n_pad, h), jnp.float32)
    sc_scatter = _make_sc_scatter(n_pad, es, h, ept, nchunk, npt_pad, chunk)

    nb_s = es // EBLK

    def make_edge_mlp(k):
        return pl.pallas_call(
            _edge_body,
            grid=(nb_s,),
            in_specs=[
                pl.BlockSpec((EBLK, h), lambda i: (i, 0)),
                pl.BlockSpec((EBLK, h), lambda i, k=k: (k * nb_s + i, 0)),
                pl.BlockSpec((1, 1, EBLK), lambda i: (i, 0, 0)),
                pl.BlockSpec((h, h), lambda i: (0, 0)),
                pl.BlockSpec((h, h), lambda i: (0, 0)),
                pl.BlockSpec((h, h), lambda i: (0, 0)),
                pl.BlockSpec((8, h), lambda i: (0, 0)),
            ],
            out_specs=[
                pl.BlockSpec((EBLK, h), lambda i: (i, 0)),
                pl.BlockSpec((EBLK, h), lambda i: (i, 0)),
            ],
            out_shape=[
                jax.ShapeDtypeStruct((es, h), jnp.float32),
                jax.ShapeDtypeStruct((es, h), jnp.float32),
            ],
        )

    gs_c = []
    for k in range(S):
        sk = lax.slice(s, (k * es,), ((k + 1) * es,)).reshape(NW, nchunk,
                                                              chunk)
        rk = lax.slice(r, (k * es,), ((k + 1) * es,))
        gs_c.append(sc_gather(g_tab, sk, rk, rtab))

    parts, eos = [], []
    for k in range(S):
        gs_k, c_k = gs_c[k]
        c3_k = c_k.reshape(nb_s, 1, EBLK)
        ne_k, eo_k = make_edge_mlp(k)(gs_k, ef, c3_k, w0c, p["edge_W1"],
                                      p["edge_W2"], evecs)
        eos.append(eo_k)
        r3_k = lax.slice(r, (k * es,), ((k + 1) * es,)).reshape(NW, nchunk,
                                                                chunk)
        parts.append(sc_scatter(ne_k, r3_k, zeros))

    eo = eos[0] if len(eos) == 1 else jnp.concatenate(eos, axis=0)
    accs = [lax.slice(part[i], (0, 0), (n, h))
            for part in parts for i in range(NC)]

    wn0 = p["node_W0"]
    wn0a, wn0b = wn0[:h], wn0[h:]
    nvecs = jnp.concatenate([
        p["node_b0"].reshape(1, h), p["node_b1"].reshape(1, h),
        p["node_b2"].reshape(1, h), p["node_g"].reshape(1, h),
        p["node_beta"].reshape(1, h), jnp.zeros((3, h), jnp.float32),
    ], axis=0)

    node_mlp = pl.pallas_call(
        _node_body,
        grid=(n // NBLK,),
        in_specs=(
            [pl.BlockSpec((NBLK, h), lambda i: (i, 0))] * (1 + len(accs))
            + [pl.BlockSpec((h, h), lambda i: (0, 0))] * 4
            + [pl.BlockSpec((8, h), lambda i: (0, 0))]
        ),
        out_specs=pl.BlockSpec((NBLK, h), lambda i: (i, 0)),
        out_shape=jax.ShapeDtypeStruct((n, h), jnp.float32),
    )
    nn = node_mlp(nf, *accs, wn0a, wn0b, p["node_W1"], p["node_W2"], nvecs)

    return nn.reshape(b, n, h), eo.reshape(b, e, h)


# speedup vs baseline: 1.5328x; 1.2025x over previous
import functools

import jax
import jax.numpy as jnp
from jax import lax
from jax.experimental import pallas as pl
from jax.experimental.pallas import tpu as pltpu
from jax.experimental.pallas import tpu_sc as plsc

NC, NS = 2, 16
NW = NC * NS
S = 1
EBLK = 8000
NBLK = 1000

def _sc_mesh():
    return plsc.VectorSubcoreMesh(core_axis_name="c", subcore_axis_name="s",
                                  num_cores=NC)


def _prep_body(nf_ref, w0a_ref, w0b_ref, b0_ref, g_ref, wr_ref):
    g_ref[...] = (
        jnp.dot(nf_ref[...], w0a_ref[...], preferred_element_type=jnp.float32)
        + b0_ref[...]
    )
    wr_ref[...] = jnp.sum(w0b_ref[...], axis=0, keepdims=True)


def _edge_body(gs_ref, ef_ref, c_ref, w0c_ref, w1_ref, w2_ref, vec_ref,
               ne_ref, eo_ref):
    ef = ef_ref[...]
    b1 = vec_ref[0:1, :]
    b2 = vec_ref[1:2, :]
    g = vec_ref[2:3, :]
    beta = vec_ref[3:4, :]
    wr = vec_ref[4:5, :]
    c2 = jnp.reshape(c_ref[...], (1, ef.shape[0]))
    couter = lax.dot_general(c2, wr, (((0,), (0,)), ((), ())),
                             preferred_element_type=jnp.float32)
    y = gs_ref[...] + couter
    y = y + jnp.dot(ef, w0c_ref[...], preferred_element_type=jnp.float32)
    y = jnp.maximum(y, 0.0)
    y = jnp.dot(y, w1_ref[...], preferred_element_type=jnp.float32) + b1
    y = jnp.maximum(y, 0.0)
    y = jnp.dot(y, w2_ref[...], preferred_element_type=jnp.float32) + b2
    mu = jnp.mean(y, axis=-1, keepdims=True)
    yc = y - mu
    var = jnp.mean(yc * yc, axis=-1, keepdims=True)
    ne = yc * lax.rsqrt(var + 1e-5) * g + beta
    ne_ref[...] = ne
    eo_ref[...] = ne + ef


def _node_body(*refs):
    nf_ref = refs[0]
    acc_refs = refs[1:-6]
    wn0a_ref, wn0b_ref, wn1_ref, wn2_ref, vec_ref, out_ref = refs[-6:]
    nf = nf_ref[...]
    acc = acc_refs[0][...]
    for a_ref in acc_refs[1:]:
        acc = acc + a_ref[...]
    b0 = vec_ref[0:1, :]
    b1 = vec_ref[1:2, :]
    b2 = vec_ref[2:3, :]
    g = vec_ref[3:4, :]
    beta = vec_ref[4:5, :]
    y = (jnp.dot(nf, wn0a_ref[...], preferred_element_type=jnp.float32)
         + jnp.dot(acc, wn0b_ref[...], preferred_element_type=jnp.float32)
         + b0)
    y = jnp.maximum(y, 0.0)
    y = jnp.dot(y, wn1_ref[...], preferred_element_type=jnp.float32) + b1
    y = jnp.maximum(y, 0.0)
    y = jnp.dot(y, wn2_ref[...], preferred_element_type=jnp.float32) + b2
    mu = jnp.mean(y, axis=-1, keepdims=True)
    yc = y - mu
    var = jnp.mean(yc * yc, axis=-1, keepdims=True)
    out_ref[...] = yc * lax.rsqrt(var + 1e-5) * g + beta + nf


def _make_sc_gather(n, e, h, ept, nchunk, CHUNK):
    @functools.partial(
        pl.kernel,
        mesh=_sc_mesh(),
        compiler_params=pltpu.CompilerParams(needs_layout_passes=False),
        out_type=(jax.ShapeDtypeStruct((e, h), jnp.float32),
                  jax.ShapeDtypeStruct((e,), jnp.float32)),
        scratch_types=[
            pltpu.VMEM((nchunk, CHUNK), jnp.int32),
            pltpu.VMEM((ept,), jnp.int32),
            pltpu.VMEM((n,), jnp.int32),
            pltpu.VMEM((ept,), jnp.float32),
        ] + [pltpu.VMEM((CHUNK, h), jnp.float32)] * 4
          + [pltpu.SemaphoreType.DMA] * 8,
    )
    def sc_gather(g_hbm, sidx_hbm, ridx_hbm, rtab_hbm, gs_out, c_out,
                  sidx_v, ridx_v, rtab_v, c_v, *bufsem):
        rows = bufsem[0:4]
        gsem = bufsem[4:8]
        wsem = bufsem[8:12]
        wid = lax.axis_index("s") * NC + lax.axis_index("c")
        ebase = wid * ept
        pltpu.sync_copy(sidx_hbm.at[wid], sidx_v)
        for bb in range(min(4, nchunk)):
            pltpu.async_copy(g_hbm.at[sidx_v.at[bb]], rows[bb], gsem[bb])
        pltpu.sync_copy(ridx_hbm.at[pl.ds(ebase, ept)], ridx_v)
        pltpu.sync_copy(rtab_hbm, rtab_v)

        def c_step(i, carry):
            idx = ridx_v[pl.ds(i * 16, 16)]
            vals = plsc.load_gather(rtab_v, [idx])
            c_v[pl.ds(i * 16, 16)] = vals.astype(jnp.float32)
            return carry

        lax.fori_loop(0, ept // 16, c_step, 0)
        pltpu.sync_copy(c_v, c_out.at[pl.ds(ebase, ept)])

        def out_at(j):
            return gs_out.at[pl.ds(ebase + j * CHUNK, CHUNK)]

        def gat(j, bb):
            return pltpu.make_async_copy(g_hbm.at[sidx_v.at[j]], rows[bb],
                                         gsem[bb])

        def wb(j, bb):
            return pltpu.make_async_copy(rows[bb], out_at(j), wsem[bb])

        def g_quad(q, carry):
            j0 = q * 4
            for bb in range(4):
                gat(j0 + bb, bb).wait()
                wb(j0 + bb, bb).start()
            for bb in range(4):
                wb(j0 + bb, bb).wait()

                @pl.when(j0 + 4 + bb < nchunk)
                def _(bb=bb):
                    gat(j0 + 4 + bb, bb).start()

            return carry

        lax.fori_loop(0, nchunk // 4, g_quad, 0)
        for t in range(nchunk % 4):
            j = (nchunk // 4) * 4 + t
            gat(j, t).wait()
            pltpu.sync_copy(rows[t], out_at(j))

    return sc_gather


def _make_sc_scatter(n_pad, e, h, ept, nchunk, npt, CHUNK):
    @functools.partial(
        pl.kernel,
        mesh=_sc_mesh(),
        compiler_params=pltpu.CompilerParams(needs_layout_passes=False),
        out_type=jax.ShapeDtypeStruct((NC, n_pad, h), jnp.float32),
        scratch_types=[
            pltpu.VMEM((nchunk, CHUNK), jnp.int32),
            pltpu.VMEM_SHARED((n_pad, h), jnp.float32),
        ] + [pltpu.VMEM((CHUNK, h), jnp.float32)] * 3
          + [pltpu.SemaphoreType.DMA] * 3,
    )
    def sc_scatter(ne_hbm, ridx_hbm, zeros_hbm, out_hbm, idx_v, acc_sh,
                   *bufsem):
        bufs = bufsem[0:3]
        lsem = bufsem[3:6]
        cid = lax.axis_index("c")
        sid = lax.axis_index("s")
        wid = sid * NC + cid
        ebase = wid * ept
        pltpu.sync_copy(ridx_hbm.at[wid], idx_v)

        def ld(j, bb):
            return pltpu.make_async_copy(
                ne_hbm.at[pl.ds(ebase + j * CHUNK, CHUNK)], bufs[bb],
                lsem[bb])

        for bb in range(min(3, nchunk)):
            ld(bb, bb).start()
        pltpu.sync_copy(zeros_hbm.at[pl.ds(sid * npt, npt)],
                        acc_sh.at[pl.ds(sid * npt, npt)])
        plsc.subcore_barrier()

        def s_trip(q, carry):
            j0 = q * 3
            for bb in range(3):
                ld(j0 + bb, bb).wait()
                pltpu.sync_copy(bufs[bb], acc_sh.at[idx_v.at[j0 + bb]],
                                add=True)

                @pl.when(j0 + 3 + bb < nchunk)
                def _(bb=bb):
                    ld(j0 + 3 + bb, bb).start()

            return carry

        lax.fori_loop(0, nchunk // 3, s_trip, 0)
        for t in range(nchunk % 3):
            j = (nchunk // 3) * 3 + t
            ld(j, t).wait()
            pltpu.sync_copy(bufs[t], acc_sh.at[idx_v.at[j]], add=True)
        plsc.subcore_barrier()
        pltpu.sync_copy(acc_sh.at[pl.ds(sid * npt, npt)],
                        out_hbm.at[cid, pl.ds(sid * npt, npt)])

    return sc_scatter


def kernel(senders, receivers, node_features, edge_features, params):
    b, n, h = node_features.shape
    e = senders.shape[1]
    es = e // S
    ept = es // NW
    chunk = 80 if ept % 80 == 0 else 40
    nchunk = ept // chunk
    npt = n // NS

    s = senders.reshape(e).astype(jnp.int32)
    r = receivers.reshape(e).astype(jnp.int32)
    nf = node_features.reshape(n, h)
    ef = edge_features.reshape(e, h)
    p = params

    w0 = p["edge_W0"]
    w0a, w0b, w0c = w0[:h], w0[h:2 * h], w0[2 * h:]

    prep = pl.pallas_call(
        _prep_body,
        grid=(n // NBLK,),
        in_specs=[
            pl.BlockSpec((NBLK, h), lambda i: (i, 0)),
            pl.BlockSpec((h, h), lambda i: (0, 0)),
            pl.BlockSpec((h, h), lambda i: (0, 0)),
            pl.BlockSpec((1, h), lambda i: (0, 0)),
        ],
        out_specs=[
            pl.BlockSpec((NBLK, h), lambda i: (i, 0)),
            pl.BlockSpec((1, h), lambda i: (0, 0)),
        ],
        out_shape=[
            jax.ShapeDtypeStruct((n, h), jnp.float32),
            jax.ShapeDtypeStruct((1, h), jnp.float32),
        ],
    )
    g_tab, wr = prep(nf, w0a, w0b, p["edge_b0"].reshape(1, h))

    rtab = r[:n]
    evecs = jnp.concatenate([
        p["edge_b1"].reshape(1, h), p["edge_b2"].reshape(1, h),
        p["edge_g"].reshape(1, h), p["edge_beta"].reshape(1, h),
        wr, jnp.zeros((3, h), jnp.float32),
    ], axis=0)

    sc_gather = _make_sc_gather(n, es, h, ept, nchunk, chunk)
    npt_pad = -(-npt // 8) * 8
    n_pad = NS * npt_pad
    zeros = jnp.zeros((n_pad, h), jnp.float32)
    sc_scatter = _make_sc_scatter(n_pad, es, h, ept, nchunk, npt_pad, chunk)

    nb_s = es // EBLK

    def make_edge_mlp(k):
        return pl.pallas_call(
            _edge_body,
            grid=(nb_s,),
            in_specs=[
                pl.BlockSpec((EBLK, h), lambda i: (i, 0)),
                pl.BlockSpec((EBLK, h), lambda i, k=k: (k * nb_s + i, 0)),
                pl.BlockSpec((1, 1, EBLK), lambda i: (i, 0, 0)),
                pl.BlockSpec((h, h), lambda i: (0, 0)),
                pl.BlockSpec((h, h), lambda i: (0, 0)),
                pl.BlockSpec((h, h), lambda i: (0, 0)),
                pl.BlockSpec((8, h), lambda i: (0, 0)),
            ],
            out_specs=[
                pl.BlockSpec((EBLK, h), lambda i: (i, 0)),
                pl.BlockSpec((EBLK, h), lambda i: (i, 0)),
            ],
            out_shape=[
                jax.ShapeDtypeStruct((es, h), jnp.float32),
                jax.ShapeDtypeStruct((es, h), jnp.float32),
            ],
        )

    gs_c = []
    for k in range(S):
        sk = lax.slice(s, (k * es,), ((k + 1) * es,)).reshape(NW, nchunk,
                                                              chunk)
        rk = lax.slice(r, (k * es,), ((k + 1) * es,))
        gs_c.append(sc_gather(g_tab, sk, rk, rtab))

    parts, eos = [], []
    for k in range(S):
        gs_k, c_k = gs_c[k]
        c3_k = c_k.reshape(nb_s, 1, EBLK)
        ne_k, eo_k = make_edge_mlp(k)(gs_k, ef, c3_k, w0c, p["edge_W1"],
                                      p["edge_W2"], evecs)
        eos.append(eo_k)
        r3_k = lax.slice(r, (k * es,), ((k + 1) * es,)).reshape(NW, nchunk,
                                                                chunk)
        parts.append(sc_scatter(ne_k, r3_k, zeros))

    eo = eos[0] if len(eos) == 1 else jnp.concatenate(eos, axis=0)
    accs = [lax.slice(part[i], (0, 0), (n, h))
            for part in parts for i in range(NC)]

    wn0 = p["node_W0"]
    wn0a, wn0b = wn0[:h], wn0[h:]
    nvecs = jnp.concatenate([
        p["node_b0"].reshape(1, h), p["node_b1"].reshape(1, h),
        p["node_b2"].reshape(1, h), p["node_g"].reshape(1, h),
        p["node_beta"].reshape(1, h), jnp.zeros((3, h), jnp.float32),
    ], axis=0)

    node_mlp = pl.pallas_call(
        _node_body,
        grid=(n // NBLK,),
        in_specs=(
            [pl.BlockSpec((NBLK, h), lambda i: (i, 0))] * (1 + len(accs))
            + [pl.BlockSpec((h, h), lambda i: (0, 0))] * 4
            + [pl.BlockSpec((8, h), lambda i: (0, 0))]
        ),
        out_specs=pl.BlockSpec((NBLK, h), lambda i: (i, 0)),
        out_shape=jax.ShapeDtypeStruct((n, h), jnp.float32),
    )
    nn = node_mlp(nf, *accs, wn0a, wn0b, p["node_W1"], p["node_W2"], nvecs)

    return nn.reshape(b, n, h), eo.reshape(b, e, h)


# speedup vs baseline: 1.5373x; 1.0030x over previous
import functools

import jax
import jax.numpy as jnp
from jax import lax
from jax.experimental import pallas as pl
from jax.experimental.pallas import tpu as pltpu
from jax.experimental.pallas import tpu_sc as plsc

NC, NS = 2, 16
NW = NC * NS
S = 1
EBLK = 8000
NBLK = 1000

def _sc_mesh():
    return plsc.VectorSubcoreMesh(core_axis_name="c", subcore_axis_name="s",
                                  num_cores=NC)


def _prep_body(nf_ref, w0a_ref, w0b_ref, b0_ref, g_ref, wr_ref):
    g_ref[...] = (
        jnp.dot(nf_ref[...], w0a_ref[...], preferred_element_type=jnp.float32)
        + b0_ref[...]
    )
    wr_ref[...] = jnp.sum(w0b_ref[...], axis=0, keepdims=True)


def _edge_body(gs_ref, ef_ref, c_ref, w0c_ref, w1_ref, w2_ref, vec_ref,
               ne_ref, eo_ref):
    ef = ef_ref[...]
    b1 = vec_ref[0:1, :]
    b2 = vec_ref[1:2, :]
    g = vec_ref[2:3, :]
    beta = vec_ref[3:4, :]
    wr = vec_ref[4:5, :]
    c2 = jnp.reshape(c_ref[...], (1, ef.shape[0]))
    couter = lax.dot_general(c2, wr, (((0,), (0,)), ((), ())),
                             preferred_element_type=jnp.float32)
    y = gs_ref[...] + couter
    y = y + jnp.dot(ef, w0c_ref[...], preferred_element_type=jnp.float32)
    y = jnp.maximum(y, 0.0)
    y = jnp.dot(y, w1_ref[...], preferred_element_type=jnp.float32) + b1
    y = jnp.maximum(y, 0.0)
    y = jnp.dot(y, w2_ref[...], preferred_element_type=jnp.float32) + b2
    mu = jnp.mean(y, axis=-1, keepdims=True)
    yc = y - mu
    var = jnp.mean(yc * yc, axis=-1, keepdims=True)
    ne = yc * lax.rsqrt(var + 1e-5) * g + beta
    ne_ref[...] = ne
    eo_ref[...] = ne + ef


def _node_body(*refs):
    nf_ref = refs[0]
    acc_refs = refs[1:-6]
    wn0a_ref, wn0b_ref, wn1_ref, wn2_ref, vec_ref, out_ref = refs[-6:]
    nf = nf_ref[...]
    acc = acc_refs[0][...]
    for a_ref in acc_refs[1:]:
        acc = acc + a_ref[...]
    b0 = vec_ref[0:1, :]
    b1 = vec_ref[1:2, :]
    b2 = vec_ref[2:3, :]
    g = vec_ref[3:4, :]
    beta = vec_ref[4:5, :]
    y = (jnp.dot(nf, wn0a_ref[...], preferred_element_type=jnp.float32)
         + jnp.dot(acc, wn0b_ref[...], preferred_element_type=jnp.float32)
         + b0)
    y = jnp.maximum(y, 0.0)
    y = jnp.dot(y, wn1_ref[...], preferred_element_type=jnp.float32) + b1
    y = jnp.maximum(y, 0.0)
    y = jnp.dot(y, wn2_ref[...], preferred_element_type=jnp.float32) + b2
    mu = jnp.mean(y, axis=-1, keepdims=True)
    yc = y - mu
    var = jnp.mean(yc * yc, axis=-1, keepdims=True)
    out_ref[...] = yc * lax.rsqrt(var + 1e-5) * g + beta + nf


def _make_sc_gather(n, e, h, ept, nchunk, CHUNK):
    @functools.partial(
        pl.kernel,
        mesh=_sc_mesh(),
        compiler_params=pltpu.CompilerParams(needs_layout_passes=False),
        out_type=(jax.ShapeDtypeStruct((e, h), jnp.float32),
                  jax.ShapeDtypeStruct((e,), jnp.float32)),
        scratch_types=[
            pltpu.VMEM((nchunk, CHUNK), jnp.int32),
            pltpu.VMEM((ept,), jnp.int32),
            pltpu.VMEM((n,), jnp.int32),
            pltpu.VMEM((ept,), jnp.float32),
        ] + [pltpu.VMEM((CHUNK, h), jnp.float32)] * 8
          + [pltpu.SemaphoreType.DMA] * 16,
    )
    def sc_gather(g_hbm, sidx_hbm, ridx_hbm, rtab_hbm, gs_out, c_out,
                  sidx_v, ridx_v, rtab_v, c_v, *bufsem):
        D = 8
        rows = bufsem[0:D]
        gsem = bufsem[D:2 * D]
        wsem = bufsem[2 * D:3 * D]
        wid = lax.axis_index("s") * NC + lax.axis_index("c")
        ebase = wid * ept
        pltpu.sync_copy(sidx_hbm.at[wid], sidx_v)
        for bb in range(min(D, nchunk)):
            pltpu.async_copy(g_hbm.at[sidx_v.at[bb]], rows[bb], gsem[bb])
        pltpu.sync_copy(ridx_hbm.at[pl.ds(ebase, ept)], ridx_v)
        pltpu.sync_copy(rtab_hbm, rtab_v)

        def c_step(i, carry):
            idx = ridx_v[pl.ds(i * 16, 16)]
            vals = plsc.load_gather(rtab_v, [idx])
            c_v[pl.ds(i * 16, 16)] = vals.astype(jnp.float32)
            return carry

        lax.fori_loop(0, ept // 16, c_step, 0)
        pltpu.sync_copy(c_v, c_out.at[pl.ds(ebase, ept)])

        def out_at(j):
            return gs_out.at[pl.ds(ebase + j * CHUNK, CHUNK)]

        def gat(j, bb):
            return pltpu.make_async_copy(g_hbm.at[sidx_v.at[j]], rows[bb],
                                         gsem[bb])

        def wb(j, bb):
            return pltpu.make_async_copy(rows[bb], out_at(j), wsem[bb])

        def g_round(q, carry):
            j0 = q * D
            for bb in range(D):
                gat(j0 + bb, bb).wait()
                wb(j0 + bb, bb).start()
            for bb in range(D):
                wb(j0 + bb, bb).wait()

                @pl.when(j0 + D + bb < nchunk)
                def _(bb=bb):
                    gat(j0 + D + bb, bb).start()

            return carry

        lax.fori_loop(0, nchunk // D, g_round, 0)
        for t in range(nchunk % D):
            j = (nchunk // D) * D + t
            gat(j, t).wait()
            pltpu.sync_copy(rows[t], out_at(j))

    return sc_gather


def _make_sc_scatter(n_pad, e, h, ept, nchunk, npt, CHUNK):
    @functools.partial(
        pl.kernel,
        mesh=_sc_mesh(),
        compiler_params=pltpu.CompilerParams(needs_layout_passes=False),
        out_type=jax.ShapeDtypeStruct((NC, n_pad, h), jnp.float32),
        scratch_types=[
            pltpu.VMEM((nchunk, CHUNK), jnp.int32),
            pltpu.VMEM_SHARED((n_pad, h), jnp.float32),
        ] + [pltpu.VMEM((CHUNK, h), jnp.float32)] * 3
          + [pltpu.SemaphoreType.DMA] * 3,
    )
    def sc_scatter(ne_hbm, ridx_hbm, zeros_hbm, out_hbm, idx_v, acc_sh,
                   *bufsem):
        bufs = bufsem[0:3]
        lsem = bufsem[3:6]
        cid = lax.axis_index("c")
        sid = lax.axis_index("s")
        wid = sid * NC + cid
        ebase = wid * ept
        pltpu.sync_copy(ridx_hbm.at[wid], idx_v)

        def ld(j, bb):
            return pltpu.make_async_copy(
                ne_hbm.at[pl.ds(ebase + j * CHUNK, CHUNK)], bufs[bb],
                lsem[bb])

        for bb in range(min(3, nchunk)):
            ld(bb, bb).start()
        pltpu.sync_copy(zeros_hbm.at[pl.ds(sid * npt, npt)],
                        acc_sh.at[pl.ds(sid * npt, npt)])
        plsc.subcore_barrier()

        def s_trip(q, carry):
            j0 = q * 3
            for bb in range(3):
                ld(j0 + bb, bb).wait()
                pltpu.sync_copy(bufs[bb], acc_sh.at[idx_v.at[j0 + bb]],
                                add=True)

                @pl.when(j0 + 3 + bb < nchunk)
                def _(bb=bb):
                    ld(j0 + 3 + bb, bb).start()

            return carry

        lax.fori_loop(0, nchunk // 3, s_trip, 0)
        for t in range(nchunk % 3):
            j = (nchunk // 3) * 3 + t
            ld(j, t).wait()
            pltpu.sync_copy(bufs[t], acc_sh.at[idx_v.at[j]], add=True)
        plsc.subcore_barrier()
        pltpu.sync_copy(acc_sh.at[pl.ds(sid * npt, npt)],
                        out_hbm.at[cid, pl.ds(sid * npt, npt)])

    return sc_scatter


def kernel(senders, receivers, node_features, edge_features, params):
    b, n, h = node_features.shape
    e = senders.shape[1]
    es = e // S
    ept = es // NW
    chunk = 80 if ept % 80 == 0 else 40
    nchunk = ept // chunk
    npt = n // NS

    s = senders.reshape(e).astype(jnp.int32)
    r = receivers.reshape(e).astype(jnp.int32)
    nf = node_features.reshape(n, h)
    ef = edge_features.reshape(e, h)
    p = params

    w0 = p["edge_W0"]
    w0a, w0b, w0c = w0[:h], w0[h:2 * h], w0[2 * h:]

    prep = pl.pallas_call(
        _prep_body,
        grid=(n // NBLK,),
        in_specs=[
            pl.BlockSpec((NBLK, h), lambda i: (i, 0)),
            pl.BlockSpec((h, h), lambda i: (0, 0)),
            pl.BlockSpec((h, h), lambda i: (0, 0)),
            pl.BlockSpec((1, h), lambda i: (0, 0)),
        ],
        out_specs=[
            pl.BlockSpec((NBLK, h), lambda i: (i, 0)),
            pl.BlockSpec((1, h), lambda i: (0, 0)),
        ],
        out_shape=[
            jax.ShapeDtypeStruct((n, h), jnp.float32),
            jax.ShapeDtypeStruct((1, h), jnp.float32),
        ],
    )
    g_tab, wr = prep(nf, w0a, w0b, p["edge_b0"].reshape(1, h))

    rtab = r[:n]
    evecs = jnp.concatenate([
        p["edge_b1"].reshape(1, h), p["edge_b2"].reshape(1, h),
        p["edge_g"].reshape(1, h), p["edge_beta"].reshape(1, h),
        wr, jnp.zeros((3, h), jnp.float32),
    ], axis=0)

    sc_gather = _make_sc_gather(n, es, h, ept, nchunk, chunk)
    npt_pad = -(-npt // 8) * 8
    n_pad = NS * npt_pad
    zeros = jnp.zeros((n_pad, h), jnp.float32)
    sc_scatter = _make_sc_scatter(n_pad, es, h, ept, nchunk, npt_pad, chunk)

    nb_s = es // EBLK

    def make_edge_mlp(k):
        return pl.pallas_call(
            _edge_body,
            grid=(nb_s,),
            in_specs=[
                pl.BlockSpec((EBLK, h), lambda i: (i, 0)),
                pl.BlockSpec((EBLK, h), lambda i, k=k: (k * nb_s + i, 0)),
                pl.BlockSpec((1, 1, EBLK), lambda i: (i, 0, 0)),
                pl.BlockSpec((h, h), lambda i: (0, 0)),
                pl.BlockSpec((h, h), lambda i: (0, 0)),
                pl.BlockSpec((h, h), lambda i: (0, 0)),
                pl.BlockSpec((8, h), lambda i: (0, 0)),
            ],
            out_specs=[
                pl.BlockSpec((EBLK, h), lambda i: (i, 0)),
                pl.BlockSpec((EBLK, h), lambda i: (i, 0)),
            ],
            out_shape=[
                jax.ShapeDtypeStruct((es, h), jnp.float32),
                jax.ShapeDtypeStruct((es, h), jnp.float32),
            ],
        )

    gs_c = []
    for k in range(S):
        sk = lax.slice(s, (k * es,), ((k + 1) * es,)).reshape(NW, nchunk,
                                                              chunk)
        rk = lax.slice(r, (k * es,), ((k + 1) * es,))
        gs_c.append(sc_gather(g_tab, sk, rk, rtab))

    parts, eos = [], []
    for k in range(S):
        gs_k, c_k = gs_c[k]
        c3_k = c_k.reshape(nb_s, 1, EBLK)
        ne_k, eo_k = make_edge_mlp(k)(gs_k, ef, c3_k, w0c, p["edge_W1"],
                                      p["edge_W2"], evecs)
        eos.append(eo_k)
        r3_k = lax.slice(r, (k * es,), ((k + 1) * es,)).reshape(NW, nchunk,
                                                                chunk)
        parts.append(sc_scatter(ne_k, r3_k, zeros))

    eo = eos[0] if len(eos) == 1 else jnp.concatenate(eos, axis=0)
    accs = [lax.slice(part[i], (0, 0), (n, h))
            for part in parts for i in range(NC)]

    wn0 = p["node_W0"]
    wn0a, wn0b = wn0[:h], wn0[h:]
    nvecs = jnp.concatenate([
        p["node_b0"].reshape(1, h), p["node_b1"].reshape(1, h),
        p["node_b2"].reshape(1, h), p["node_g"].reshape(1, h),
        p["node_beta"].reshape(1, h), jnp.zeros((3, h), jnp.float32),
    ], axis=0)

    node_mlp = pl.pallas_call(
        _node_body,
        grid=(n // NBLK,),
        in_specs=(
            [pl.BlockSpec((NBLK, h), lambda i: (i, 0))] * (1 + len(accs))
            + [pl.BlockSpec((h, h), lambda i: (0, 0))] * 4
            + [pl.BlockSpec((8, h), lambda i: (0, 0))]
        ),
        out_specs=pl.BlockSpec((NBLK, h), lambda i: (i, 0)),
        out_shape=jax.ShapeDtypeStruct((n, h), jnp.float32),
    )
    nn = node_mlp(nf, *accs, wn0a, wn0b, p["node_W1"], p["node_W2"], nvecs)

    return nn.reshape(b, n, h), eo.reshape(b, e, h)


# speedup vs baseline: 1.5554x; 1.0118x over previous
import functools

import jax
import jax.numpy as jnp
from jax import lax
from jax.experimental import pallas as pl
from jax.experimental.pallas import tpu as pltpu
from jax.experimental.pallas import tpu_sc as plsc

NC, NS = 2, 16
NW = NC * NS
S = 1
EBLK = 10000
NBLK = 1000

def _sc_mesh():
    return plsc.VectorSubcoreMesh(core_axis_name="c", subcore_axis_name="s",
                                  num_cores=NC)


def _prep_body(nf_ref, w0a_ref, w0b_ref, b0_ref, g_ref, wr_ref):
    g_ref[...] = (
        jnp.dot(nf_ref[...], w0a_ref[...], preferred_element_type=jnp.float32)
        + b0_ref[...]
    )
    wr_ref[...] = jnp.sum(w0b_ref[...], axis=0, keepdims=True)


def _edge_body(gs_ref, ef_ref, c_ref, w0c_ref, w1_ref, w2_ref, vec_ref,
               ne_ref, eo_ref):
    ef = ef_ref[...]
    b1 = vec_ref[0:1, :]
    b2 = vec_ref[1:2, :]
    g = vec_ref[2:3, :]
    beta = vec_ref[3:4, :]
    wr = vec_ref[4:5, :]
    c2 = jnp.reshape(c_ref[...], (1, ef.shape[0]))
    couter = lax.dot_general(c2, wr, (((0,), (0,)), ((), ())),
                             preferred_element_type=jnp.float32)
    y = gs_ref[...] + couter
    y = y + jnp.dot(ef, w0c_ref[...], preferred_element_type=jnp.float32)
    y = jnp.maximum(y, 0.0)
    y = jnp.dot(y, w1_ref[...], preferred_element_type=jnp.float32) + b1
    y = jnp.maximum(y, 0.0)
    y = jnp.dot(y, w2_ref[...], preferred_element_type=jnp.float32) + b2
    mu = jnp.mean(y, axis=-1, keepdims=True)
    yc = y - mu
    var = jnp.mean(yc * yc, axis=-1, keepdims=True)
    ne = yc * lax.rsqrt(var + 1e-5) * g + beta
    ne_ref[...] = ne
    eo_ref[...] = ne + ef


def _node_body(*refs):
    nf_ref = refs[0]
    acc_refs = refs[1:-6]
    wn0a_ref, wn0b_ref, wn1_ref, wn2_ref, vec_ref, out_ref = refs[-6:]
    nf = nf_ref[...]
    acc = acc_refs[0][...]
    for a_ref in acc_refs[1:]:
        acc = acc + a_ref[...]
    b0 = vec_ref[0:1, :]
    b1 = vec_ref[1:2, :]
    b2 = vec_ref[2:3, :]
    g = vec_ref[3:4, :]
    beta = vec_ref[4:5, :]
    y = (jnp.dot(nf, wn0a_ref[...], preferred_element_type=jnp.float32)
         + jnp.dot(acc, wn0b_ref[...], preferred_element_type=jnp.float32)
         + b0)
    y = jnp.maximum(y, 0.0)
    y = jnp.dot(y, wn1_ref[...], preferred_element_type=jnp.float32) + b1
    y = jnp.maximum(y, 0.0)
    y = jnp.dot(y, wn2_ref[...], preferred_element_type=jnp.float32) + b2
    mu = jnp.mean(y, axis=-1, keepdims=True)
    yc = y - mu
    var = jnp.mean(yc * yc, axis=-1, keepdims=True)
    out_ref[...] = yc * lax.rsqrt(var + 1e-5) * g + beta + nf


def _make_sc_gather(n, e, h, ept, nchunk, CHUNK):
    @functools.partial(
        pl.kernel,
        mesh=_sc_mesh(),
        compiler_params=pltpu.CompilerParams(needs_layout_passes=False),
        out_type=(jax.ShapeDtypeStruct((e, h), jnp.float32),
                  jax.ShapeDtypeStruct((e,), jnp.float32)),
        scratch_types=[
            pltpu.VMEM((nchunk, CHUNK), jnp.int32),
            pltpu.VMEM((ept,), jnp.int32),
            pltpu.VMEM((n,), jnp.int32),
            pltpu.VMEM((ept,), jnp.float32),
        ] + [pltpu.VMEM((CHUNK, h), jnp.float32)] * 8
          + [pltpu.SemaphoreType.DMA] * 16,
    )
    def sc_gather(g_hbm, sidx_hbm, ridx_hbm, rtab_hbm, gs_out, c_out,
                  sidx_v, ridx_v, rtab_v, c_v, *bufsem):
        D = 8
        rows = bufsem[0:D]
        gsem = bufsem[D:2 * D]
        wsem = bufsem[2 * D:3 * D]
        wid = lax.axis_index("s") * NC + lax.axis_index("c")
        ebase = wid * ept
        pltpu.sync_copy(sidx_hbm.at[wid], sidx_v)
        for bb in range(min(D, nchunk)):
            pltpu.async_copy(g_hbm.at[sidx_v.at[bb]], rows[bb], gsem[bb])
        pltpu.sync_copy(ridx_hbm.at[pl.ds(ebase, ept)], ridx_v)
        pltpu.sync_copy(rtab_hbm, rtab_v)

        def c_step(i, carry):
            idx = ridx_v[pl.ds(i * 16, 16)]
            vals = plsc.load_gather(rtab_v, [idx])
            c_v[pl.ds(i * 16, 16)] = vals.astype(jnp.float32)
            return carry

        lax.fori_loop(0, ept // 16, c_step, 0)
        pltpu.sync_copy(c_v, c_out.at[pl.ds(ebase, ept)])

        def out_at(j):
            return gs_out.at[pl.ds(ebase + j * CHUNK, CHUNK)]

        def gat(j, bb):
            return pltpu.make_async_copy(g_hbm.at[sidx_v.at[j]], rows[bb],
                                         gsem[bb])

        def wb(j, bb):
            return pltpu.make_async_copy(rows[bb], out_at(j), wsem[bb])

        def g_round(q, carry):
            j0 = q * D
            for bb in range(D):
                gat(j0 + bb, bb).wait()
                wb(j0 + bb, bb).start()
            for bb in range(D):
                wb(j0 + bb, bb).wait()

                @pl.when(j0 + D + bb < nchunk)
                def _(bb=bb):
                    gat(j0 + D + bb, bb).start()

            return carry

        lax.fori_loop(0, nchunk // D, g_round, 0)
        for t in range(nchunk % D):
            j = (nchunk // D) * D + t
            gat(j, t).wait()
            pltpu.sync_copy(rows[t], out_at(j))

    return sc_gather


def _make_sc_scatter(n_pad, e, h, ept, nchunk, npt, CHUNK):
    @functools.partial(
        pl.kernel,
        mesh=_sc_mesh(),
        compiler_params=pltpu.CompilerParams(needs_layout_passes=False),
        out_type=jax.ShapeDtypeStruct((NC, n_pad, h), jnp.float32),
        scratch_types=[
            pltpu.VMEM((nchunk, CHUNK), jnp.int32),
            pltpu.VMEM_SHARED((n_pad, h), jnp.float32),
        ] + [pltpu.VMEM((CHUNK, h), jnp.float32)] * 3
          + [pltpu.SemaphoreType.DMA] * 3,
    )
    def sc_scatter(ne_hbm, ridx_hbm, zeros_hbm, out_hbm, idx_v, acc_sh,
                   *bufsem):
        bufs = bufsem[0:3]
        lsem = bufsem[3:6]
        cid = lax.axis_index("c")
        sid = lax.axis_index("s")
        wid = sid * NC + cid
        ebase = wid * ept
        pltpu.sync_copy(ridx_hbm.at[wid], idx_v)

        def ld(j, bb):
            return pltpu.make_async_copy(
                ne_hbm.at[pl.ds(ebase + j * CHUNK, CHUNK)], bufs[bb],
                lsem[bb])

        for bb in range(min(3, nchunk)):
            ld(bb, bb).start()
        pltpu.sync_copy(zeros_hbm.at[pl.ds(sid * npt, npt)],
                        acc_sh.at[pl.ds(sid * npt, npt)])
        plsc.subcore_barrier()

        def s_trip(q, carry):
            j0 = q * 3
            for bb in range(3):
                ld(j0 + bb, bb).wait()
                pltpu.sync_copy(bufs[bb], acc_sh.at[idx_v.at[j0 + bb]],
                                add=True)

                @pl.when(j0 + 3 + bb < nchunk)
                def _(bb=bb):
                    ld(j0 + 3 + bb, bb).start()

            return carry

        lax.fori_loop(0, nchunk // 3, s_trip, 0)
        for t in range(nchunk % 3):
            j = (nchunk // 3) * 3 + t
            ld(j, t).wait()
            pltpu.sync_copy(bufs[t], acc_sh.at[idx_v.at[j]], add=True)
        plsc.subcore_barrier()
        pltpu.sync_copy(acc_sh.at[pl.ds(sid * npt, npt)],
                        out_hbm.at[cid, pl.ds(sid * npt, npt)])

    return sc_scatter


def kernel(senders, receivers, node_features, edge_features, params):
    b, n, h = node_features.shape
    e = senders.shape[1]
    es = e // S
    ept = es // NW
    chunk = 80 if ept % 80 == 0 else 40
    nchunk = ept // chunk
    npt = n // NS

    s = senders.reshape(e).astype(jnp.int32)
    r = receivers.reshape(e).astype(jnp.int32)
    nf = node_features.reshape(n, h)
    ef = edge_features.reshape(e, h)
    p = params

    w0 = p["edge_W0"]
    w0a, w0b, w0c = w0[:h], w0[h:2 * h], w0[2 * h:]

    prep = pl.pallas_call(
        _prep_body,
        grid=(n // NBLK,),
        in_specs=[
            pl.BlockSpec((NBLK, h), lambda i: (i, 0)),
            pl.BlockSpec((h, h), lambda i: (0, 0)),
            pl.BlockSpec((h, h), lambda i: (0, 0)),
            pl.BlockSpec((1, h), lambda i: (0, 0)),
        ],
        out_specs=[
            pl.BlockSpec((NBLK, h), lambda i: (i, 0)),
            pl.BlockSpec((1, h), lambda i: (0, 0)),
        ],
        out_shape=[
            jax.ShapeDtypeStruct((n, h), jnp.float32),
            jax.ShapeDtypeStruct((1, h), jnp.float32),
        ],
    )
    g_tab, wr = prep(nf, w0a, w0b, p["edge_b0"].reshape(1, h))

    rtab = r[:n]
    evecs = jnp.concatenate([
        p["edge_b1"].reshape(1, h), p["edge_b2"].reshape(1, h),
        p["edge_g"].reshape(1, h), p["edge_beta"].reshape(1, h),
        wr, jnp.zeros((3, h), jnp.float32),
    ], axis=0)

    sc_gather = _make_sc_gather(n, es, h, ept, nchunk, chunk)
    npt_pad = -(-npt // 8) * 8
    n_pad = NS * npt_pad
    zeros = jnp.zeros((n_pad, h), jnp.float32)
    sc_scatter = _make_sc_scatter(n_pad, es, h, ept, nchunk, npt_pad, chunk)

    nb_s = es // EBLK

    def make_edge_mlp(k):
        return pl.pallas_call(
            _edge_body,
            grid=(nb_s,),
            in_specs=[
                pl.BlockSpec((EBLK, h), lambda i: (i, 0)),
                pl.BlockSpec((EBLK, h), lambda i, k=k: (k * nb_s + i, 0)),
                pl.BlockSpec((1, 1, EBLK), lambda i: (i, 0, 0)),
                pl.BlockSpec((h, h), lambda i: (0, 0)),
                pl.BlockSpec((h, h), lambda i: (0, 0)),
                pl.BlockSpec((h, h), lambda i: (0, 0)),
                pl.BlockSpec((8, h), lambda i: (0, 0)),
            ],
            out_specs=[
                pl.BlockSpec((EBLK, h), lambda i: (i, 0)),
                pl.BlockSpec((EBLK, h), lambda i: (i, 0)),
            ],
            out_shape=[
                jax.ShapeDtypeStruct((es, h), jnp.float32),
                jax.ShapeDtypeStruct((es, h), jnp.float32),
            ],
        )

    gs_c = []
    for k in range(S):
        sk = lax.slice(s, (k * es,), ((k + 1) * es,)).reshape(NW, nchunk,
                                                              chunk)
        rk = lax.slice(r, (k * es,), ((k + 1) * es,))
        gs_c.append(sc_gather(g_tab, sk, rk, rtab))

    parts, eos = [], []
    for k in range(S):
        gs_k, c_k = gs_c[k]
        c3_k = c_k.reshape(nb_s, 1, EBLK)
        ne_k, eo_k = make_edge_mlp(k)(gs_k, ef, c3_k, w0c, p["edge_W1"],
                                      p["edge_W2"], evecs)
        eos.append(eo_k)
        r3_k = lax.slice(r, (k * es,), ((k + 1) * es,)).reshape(NW, nchunk,
                                                                chunk)
        parts.append(sc_scatter(ne_k, r3_k, zeros))

    eo = eos[0] if len(eos) == 1 else jnp.concatenate(eos, axis=0)
    accs = [lax.slice(part[i], (0, 0), (n, h))
            for part in parts for i in range(NC)]

    wn0 = p["node_W0"]
    wn0a, wn0b = wn0[:h], wn0[h:]
    nvecs = jnp.concatenate([
        p["node_b0"].reshape(1, h), p["node_b1"].reshape(1, h),
        p["node_b2"].reshape(1, h), p["node_g"].reshape(1, h),
        p["node_beta"].reshape(1, h), jnp.zeros((3, h), jnp.float32),
    ], axis=0)

    node_mlp = pl.pallas_call(
        _node_body,
        grid=(n // NBLK,),
        in_specs=(
            [pl.BlockSpec((NBLK, h), lambda i: (i, 0))] * (1 + len(accs))
            + [pl.BlockSpec((h, h), lambda i: (0, 0))] * 4
            + [pl.BlockSpec((8, h), lambda i: (0, 0))]
        ),
        out_specs=pl.BlockSpec((NBLK, h), lambda i: (i, 0)),
        out_shape=jax.ShapeDtypeStruct((n, h), jnp.float32),
    )
    nn = node_mlp(nf, *accs, wn0a, wn0b, p["node_W1"], p["node_W2"], nvecs)

    return nn.reshape(b, n, h), eo.reshape(b, e, h)


# speedup vs baseline: 1.5662x; 1.0069x over previous
import functools

import jax
import jax.numpy as jnp
from jax import lax
from jax.experimental import pallas as pl
from jax.experimental.pallas import tpu as pltpu
from jax.experimental.pallas import tpu_sc as plsc

NC, NS = 2, 16
NW = NC * NS
S = 1
EBLK = 10000
NBLK = 2000

def _sc_mesh():
    return plsc.VectorSubcoreMesh(core_axis_name="c", subcore_axis_name="s",
                                  num_cores=NC)


def _prep_body(nf_ref, w0a_ref, w0b_ref, b0_ref, g_ref, wr_ref):
    g_ref[...] = (
        jnp.dot(nf_ref[...], w0a_ref[...], preferred_element_type=jnp.float32)
        + b0_ref[...]
    )
    wr_ref[...] = jnp.sum(w0b_ref[...], axis=0, keepdims=True)


def _edge_body(gs_ref, ef_ref, c_ref, w0c_ref, w1_ref, w2_ref, vec_ref,
               ne_ref, eo_ref):
    ef = ef_ref[...]
    b1 = vec_ref[0:1, :]
    b2 = vec_ref[1:2, :]
    g = vec_ref[2:3, :]
    beta = vec_ref[3:4, :]
    wr = vec_ref[4:5, :]
    c2 = jnp.reshape(c_ref[...], (1, ef.shape[0]))
    couter = lax.dot_general(c2, wr, (((0,), (0,)), ((), ())),
                             preferred_element_type=jnp.float32)
    y = gs_ref[...] + couter
    y = y + jnp.dot(ef, w0c_ref[...], preferred_element_type=jnp.float32)
    y = jnp.maximum(y, 0.0)
    y = jnp.dot(y, w1_ref[...], preferred_element_type=jnp.float32) + b1
    y = jnp.maximum(y, 0.0)
    y = jnp.dot(y, w2_ref[...], preferred_element_type=jnp.float32) + b2
    mu = jnp.mean(y, axis=-1, keepdims=True)
    yc = y - mu
    var = jnp.mean(yc * yc, axis=-1, keepdims=True)
    ne = yc * lax.rsqrt(var + 1e-5) * g + beta
    ne_ref[...] = ne
    eo_ref[...] = ne + ef


def _node_body(*refs):
    nf_ref = refs[0]
    acc_refs = refs[1:-6]
    wn0a_ref, wn0b_ref, wn1_ref, wn2_ref, vec_ref, out_ref = refs[-6:]
    nf = nf_ref[...]
    acc = acc_refs[0][...]
    for a_ref in acc_refs[1:]:
        acc = acc + a_ref[...]
    b0 = vec_ref[0:1, :]
    b1 = vec_ref[1:2, :]
    b2 = vec_ref[2:3, :]
    g = vec_ref[3:4, :]
    beta = vec_ref[4:5, :]
    y = (jnp.dot(nf, wn0a_ref[...], preferred_element_type=jnp.float32)
         + jnp.dot(acc, wn0b_ref[...], preferred_element_type=jnp.float32)
         + b0)
    y = jnp.maximum(y, 0.0)
    y = jnp.dot(y, wn1_ref[...], preferred_element_type=jnp.float32) + b1
    y = jnp.maximum(y, 0.0)
    y = jnp.dot(y, wn2_ref[...], preferred_element_type=jnp.float32) + b2
    mu = jnp.mean(y, axis=-1, keepdims=True)
    yc = y - mu
    var = jnp.mean(yc * yc, axis=-1, keepdims=True)
    out_ref[...] = yc * lax.rsqrt(var + 1e-5) * g + beta + nf


def _make_sc_gather(n, e, h, ept, nchunk, CHUNK):
    @functools.partial(
        pl.kernel,
        mesh=_sc_mesh(),
        compiler_params=pltpu.CompilerParams(needs_layout_passes=False),
        out_type=(jax.ShapeDtypeStruct((e, h), jnp.float32),
                  jax.ShapeDtypeStruct((e,), jnp.float32)),
        scratch_types=[
            pltpu.VMEM((nchunk, CHUNK), jnp.int32),
            pltpu.VMEM((ept,), jnp.int32),
            pltpu.VMEM((n,), jnp.int32),
            pltpu.VMEM((ept,), jnp.float32),
        ] + [pltpu.VMEM((CHUNK, h), jnp.float32)] * 8
          + [pltpu.SemaphoreType.DMA] * 16,
    )
    def sc_gather(g_hbm, sidx_hbm, ridx_hbm, rtab_hbm, gs_out, c_out,
                  sidx_v, ridx_v, rtab_v, c_v, *bufsem):
        D = 8
        rows = bufsem[0:D]
        gsem = bufsem[D:2 * D]
        wsem = bufsem[2 * D:3 * D]
        wid = lax.axis_index("s") * NC + lax.axis_index("c")
        ebase = wid * ept
        pltpu.sync_copy(sidx_hbm.at[wid], sidx_v)
        for bb in range(min(D, nchunk)):
            pltpu.async_copy(g_hbm.at[sidx_v.at[bb]], rows[bb], gsem[bb])
        pltpu.sync_copy(ridx_hbm.at[pl.ds(ebase, ept)], ridx_v)
        pltpu.sync_copy(rtab_hbm, rtab_v)

        def c_step(i, carry):
            idx = ridx_v[pl.ds(i * 16, 16)]
            vals = plsc.load_gather(rtab_v, [idx])
            c_v[pl.ds(i * 16, 16)] = vals.astype(jnp.float32)
            return carry

        lax.fori_loop(0, ept // 16, c_step, 0)
        pltpu.sync_copy(c_v, c_out.at[pl.ds(ebase, ept)])

        def out_at(j):
            return gs_out.at[pl.ds(ebase + j * CHUNK, CHUNK)]

        def gat(j, bb):
            return pltpu.make_async_copy(g_hbm.at[sidx_v.at[j]], rows[bb],
                                         gsem[bb])

        def wb(j, bb):
            return pltpu.make_async_copy(rows[bb], out_at(j), wsem[bb])

        def g_round(q, carry):
            j0 = q * D
            for bb in range(D):
                gat(j0 + bb, bb).wait()
                wb(j0 + bb, bb).start()
            for bb in range(D):
                wb(j0 + bb, bb).wait()

                @pl.when(j0 + D + bb < nchunk)
                def _(bb=bb):
                    gat(j0 + D + bb, bb).start()

            return carry

        lax.fori_loop(0, nchunk // D, g_round, 0)
        for t in range(nchunk % D):
            j = (nchunk // D) * D + t
            gat(j, t).wait()
            pltpu.sync_copy(rows[t], out_at(j))

    return sc_gather


def _make_sc_scatter(n_pad, e, h, ept, nchunk, npt, CHUNK):
    @functools.partial(
        pl.kernel,
        mesh=_sc_mesh(),
        compiler_params=pltpu.CompilerParams(needs_layout_passes=False),
        out_type=jax.ShapeDtypeStruct((NC, n_pad, h), jnp.float32),
        scratch_types=[
            pltpu.VMEM((nchunk, CHUNK), jnp.int32),
            pltpu.VMEM_SHARED((n_pad, h), jnp.float32),
        ] + [pltpu.VMEM((CHUNK, h), jnp.float32)] * 3
          + [pltpu.SemaphoreType.DMA] * 3,
    )
    def sc_scatter(ne_hbm, ridx_hbm, zeros_hbm, out_hbm, idx_v, acc_sh,
                   *bufsem):
        bufs = bufsem[0:3]
        lsem = bufsem[3:6]
        cid = lax.axis_index("c")
        sid = lax.axis_index("s")
        wid = sid * NC + cid
        ebase = wid * ept
        pltpu.sync_copy(ridx_hbm.at[wid], idx_v)

        def ld(j, bb):
            return pltpu.make_async_copy(
                ne_hbm.at[pl.ds(ebase + j * CHUNK, CHUNK)], bufs[bb],
                lsem[bb])

        for bb in range(min(3, nchunk)):
            ld(bb, bb).start()
        pltpu.sync_copy(zeros_hbm.at[pl.ds(sid * npt, npt)],
                        acc_sh.at[pl.ds(sid * npt, npt)])
        plsc.subcore_barrier()

        def s_trip(q, carry):
            j0 = q * 3
            for bb in range(3):
                ld(j0 + bb, bb).wait()
                pltpu.sync_copy(bufs[bb], acc_sh.at[idx_v.at[j0 + bb]],
                                add=True)

                @pl.when(j0 + 3 + bb < nchunk)
                def _(bb=bb):
                    ld(j0 + 3 + bb, bb).start()

            return carry

        lax.fori_loop(0, nchunk // 3, s_trip, 0)
        for t in range(nchunk % 3):
            j = (nchunk // 3) * 3 + t
            ld(j, t).wait()
            pltpu.sync_copy(bufs[t], acc_sh.at[idx_v.at[j]], add=True)
        plsc.subcore_barrier()
        pltpu.sync_copy(acc_sh.at[pl.ds(sid * npt, npt)],
                        out_hbm.at[cid, pl.ds(sid * npt, npt)])

    return sc_scatter


def kernel(senders, receivers, node_features, edge_features, params):
    b, n, h = node_features.shape
    e = senders.shape[1]
    es = e // S
    ept = es // NW
    chunk = 80 if ept % 80 == 0 else 40
    nchunk = ept // chunk
    npt = n // NS

    s = senders.reshape(e).astype(jnp.int32)
    r = receivers.reshape(e).astype(jnp.int32)
    nf = node_features.reshape(n, h)
    ef = edge_features.reshape(e, h)
    p = params

    w0 = p["edge_W0"]
    w0a, w0b, w0c = w0[:h], w0[h:2 * h], w0[2 * h:]

    prep = pl.pallas_call(
        _prep_body,
        grid=(n // NBLK,),
        in_specs=[
            pl.BlockSpec((NBLK, h), lambda i: (i, 0)),
            pl.BlockSpec((h, h), lambda i: (0, 0)),
            pl.BlockSpec((h, h), lambda i: (0, 0)),
            pl.BlockSpec((1, h), lambda i: (0, 0)),
        ],
        out_specs=[
            pl.BlockSpec((NBLK, h), lambda i: (i, 0)),
            pl.BlockSpec((1, h), lambda i: (0, 0)),
        ],
        out_shape=[
            jax.ShapeDtypeStruct((n, h), jnp.float32),
            jax.ShapeDtypeStruct((1, h), jnp.float32),
        ],
    )
    g_tab, wr = prep(nf, w0a, w0b, p["edge_b0"].reshape(1, h))

    rtab = r[:n]
    evecs = jnp.concatenate([
        p["edge_b1"].reshape(1, h), p["edge_b2"].reshape(1, h),
        p["edge_g"].reshape(1, h), p["edge_beta"].reshape(1, h),
        wr, jnp.zeros((3, h), jnp.float32),
    ], axis=0)

    sc_gather = _make_sc_gather(n, es, h, ept, nchunk, chunk)
    npt_pad = -(-npt // 8) * 8
    n_pad = NS * npt_pad
    zeros = jnp.zeros((n_pad, h), jnp.float32)
    sc_scatter = _make_sc_scatter(n_pad, es, h, ept, nchunk, npt_pad, chunk)

    nb_s = es // EBLK

    def make_edge_mlp(k):
        return pl.pallas_call(
            _edge_body,
            grid=(nb_s,),
            in_specs=[
                pl.BlockSpec((EBLK, h), lambda i: (i, 0)),
                pl.BlockSpec((EBLK, h), lambda i, k=k: (k * nb_s + i, 0)),
                pl.BlockSpec((1, 1, EBLK), lambda i: (i, 0, 0)),
                pl.BlockSpec((h, h), lambda i: (0, 0)),
                pl.BlockSpec((h, h), lambda i: (0, 0)),
                pl.BlockSpec((h, h), lambda i: (0, 0)),
                pl.BlockSpec((8, h), lambda i: (0, 0)),
            ],
            out_specs=[
                pl.BlockSpec((EBLK, h), lambda i: (i, 0)),
                pl.BlockSpec((EBLK, h), lambda i: (i, 0)),
            ],
            out_shape=[
                jax.ShapeDtypeStruct((es, h), jnp.float32),
                jax.ShapeDtypeStruct((es, h), jnp.float32),
            ],
        )

    gs_c = []
    for k in range(S):
        sk = lax.slice(s, (k * es,), ((k + 1) * es,)).reshape(NW, nchunk,
                                                              chunk)
        rk = lax.slice(r, (k * es,), ((k + 1) * es,))
        gs_c.append(sc_gather(g_tab, sk, rk, rtab))

    parts, eos = [], []
    for k in range(S):
        gs_k, c_k = gs_c[k]
        c3_k = c_k.reshape(nb_s, 1, EBLK)
        ne_k, eo_k = make_edge_mlp(k)(gs_k, ef, c3_k, w0c, p["edge_W1"],
                                      p["edge_W2"], evecs)
        eos.append(eo_k)
        r3_k = lax.slice(r, (k * es,), ((k + 1) * es,)).reshape(NW, nchunk,
                                                                chunk)
        parts.append(sc_scatter(ne_k, r3_k, zeros))

    eo = eos[0] if len(eos) == 1 else jnp.concatenate(eos, axis=0)
    accs = [lax.slice(part[i], (0, 0), (n, h))
            for part in parts for i in range(NC)]

    wn0 = p["node_W0"]
    wn0a, wn0b = wn0[:h], wn0[h:]
    nvecs = jnp.concatenate([
        p["node_b0"].reshape(1, h), p["node_b1"].reshape(1, h),
        p["node_b2"].reshape(1, h), p["node_g"].reshape(1, h),
        p["node_beta"].reshape(1, h), jnp.zeros((3, h), jnp.float32),
    ], axis=0)

    node_mlp = pl.pallas_call(
        _node_body,
        grid=(n // NBLK,),
        in_specs=(
            [pl.BlockSpec((NBLK, h), lambda i: (i, 0))] * (1 + len(accs))
            + [pl.BlockSpec((h, h), lambda i: (0, 0))] * 4
            + [pl.BlockSpec((8, h), lambda i: (0, 0))]
        ),
        out_specs=pl.BlockSpec((NBLK, h), lambda i: (i, 0)),
        out_shape=jax.ShapeDtypeStruct((n, h), jnp.float32),
    )
    nn = node_mlp(nf, *accs, wn0a, wn0b, p["node_W1"], p["node_W2"], nvecs)

    return nn.reshape(b, n, h), eo.reshape(b, e, h)
